# baseline (speedup 1.0000x reference)
import functools

import jax
import jax.numpy as jnp
from jax.experimental import pallas as pl
from jax.experimental.pallas import tpu as pltpu

_CHUNK = 512
_OBLK = 512
_NCHUNK = 1024 // _CHUNK
_NOUT = 1024 // _OBLK


def _contract(a, b, adim, bdim):
    return jax.lax.dot_general(
        a, b, (((adim,), (bdim,)), ((), ())),
        preferred_element_type=jnp.float32)


def _fused_kernel(lt_ref, y_ref, w1l_ref, b1_ref, w1r_ref, w2l_ref, b2_ref,
                  w2r_ref, wg_ref, bg_ref, out_ref,
                  ltbf_s, g_s, omega_s, sw_col_s, sw_row_s):
    i = pl.program_id(0)

    @pl.when(i < _NCHUNK)
    def _gram_step():
        chunk = lt_ref[...].astype(jnp.bfloat16)
        ltbf_s[pl.ds(i * _CHUNK, _CHUNK), :] = chunk
        part = _contract(chunk, chunk, 0, 0)
        if _NCHUNK == 1:
            g_s[...] = part
        else:
            @pl.when(i == 0)
            def _():
                g_s[...] = part

            @pl.when(i > 0)
            def _():
                g_s[...] += part

    @pl.when(i == _NCHUNK - 1)
    def _omega_step():
        y = y_ref[...]
        a = (g_s[...] > 0.5).astype(jnp.float32)
        deg = jnp.maximum(jnp.sum(a, axis=1, keepdims=True), 1.0)

        s1 = _contract(a, y, 0, 0)
        mean1 = s1 / deg
        h = _contract(mean1, w1l_ref[...], 1, 1) + b1_ref[...] \
            + _contract(y, w1r_ref[...], 1, 1)
        h = jnp.maximum(h, 0.0)

        s2 = _contract(a, h, 0, 0)
        mean2 = s2 / deg
        y2 = _contract(mean2, w2l_ref[...], 1, 1) + b2_ref[...] \
            + _contract(h, w2r_ref[...], 1, 1)

        raw = jnp.tanh(_contract(wg_ref[...], y2, 1, 1) + bg_ref[...])
        wmin = jnp.min(raw)
        wmax = jnp.max(raw)
        span = wmax - wmin
        degen = jnp.abs(span) < 1e-8
        norm = (raw - wmin) / jnp.where(degen, 1.0, span)
        omega_bf = jnp.where(degen, 0.5, norm).astype(jnp.bfloat16)
        omega_s[...] = omega_bf

        ltbf = ltbf_s[...]
        lw_all = ltbf * omega_bf
        sw_col_s[...] = jnp.sum(lw_all.astype(jnp.float32), axis=1,
                                keepdims=True)
        sw_row_s[...] = _contract(omega_bf, ltbf, 1, 1)

    @pl.when(i >= _NCHUNK)
    def _tile_step():
        t = i - _NCHUNK
        rows = pl.ds(t * _OBLK, _OBLK)
        lw = ltbf_s[rows, :] * omega_s[...]
        num = _contract(lw, ltbf_s[...], 1, 1)
        den = sw_col_s[rows, :] + sw_row_s[...]
        den = jnp.where(jnp.abs(den) < 1e-8, 1.0, den)
        out_ref[...] = num / den


@functools.partial(jax.jit, static_argnames=())
def kernel(l_t, y_all_labels, W1l, b1, W1r, W2l, b2, W2r, Wg, bg):
    B, L = l_t.shape
    D = y_all_labels.shape[1]
    H = W1l.shape[0]
    O = W2l.shape[0]
    whole = lambda shape: pl.BlockSpec(shape, lambda i: (0,) * len(shape))
    return pl.pallas_call(
        _fused_kernel,
        grid=(_NCHUNK + _NOUT,),
        in_specs=[
            pl.BlockSpec((_CHUNK, L),
                         lambda i: (jnp.minimum(i, _NCHUNK - 1), 0)),
            whole((L, D)), whole((H, D)), whole((1, H)),
            whole((H, D)), whole((O, H)), whole((1, O)), whole((O, H)),
            whole((1, O)), whole((1, 1)),
        ],
        out_specs=pl.BlockSpec(
            (_OBLK, B), lambda i: (jnp.maximum(i - _NCHUNK, 0), 0)),
        out_shape=jax.ShapeDtypeStruct((B, B), jnp.float32),
        scratch_shapes=[
            pltpu.VMEM((B, L), jnp.bfloat16),
            pltpu.VMEM((L, L), jnp.float32),
            pltpu.VMEM((1, L), jnp.bfloat16),
            pltpu.VMEM((B, 1), jnp.float32),
            pltpu.VMEM((1, B), jnp.float32),
        ],
    )(l_t, y_all_labels, W1l, b1.reshape(1, -1), W1r, W2l,
      b2.reshape(1, -1), W2r, Wg, bg.reshape(1, 1))
